# baseline (speedup 1.0000x reference)
import functools

import jax
import jax.numpy as jnp
from jax.experimental import pallas as pl
from jax.experimental.pallas import tpu as pltpu

_LANES = 128
_N_SUB = 1


def _bnneck_kernel(x_ref, w_ref, gamma_ref, beta_ref, o_ref, x_asm, *, n):
    c_in = x_ref.shape[1] * _LANES

    @pl.when(pl.program_id(1) == 0)
    def _cache_x():
        x_asm[...] = x_ref[...].astype(jnp.bfloat16).reshape(n, c_in)

    wk = w_ref[...].astype(jnp.bfloat16).reshape(w_ref.shape[0], c_in)
    y = jax.lax.dot_general(
        x_asm[...], wk, dimension_numbers=(((1,), (1,)), ((), ())),
        preferred_element_type=jnp.float32)
    inv_n = 1.0 / float(n)
    mean = jnp.sum(y, axis=0, keepdims=True) * inv_n
    diff = y - mean
    var = jnp.sum(diff * diff, axis=0, keepdims=True) * inv_n
    z = diff * jax.lax.rsqrt(var + 1e-5)
    z = z * gamma_ref[...] + beta_ref[...]
    o_ref[...] = jnp.where(z >= 0, z, 0.25 * z)


def kernel(x, weight, gamma, beta):
    n, c_in, h, w_sp = x.shape
    assert h == 1 and w_sp == 1
    c_out = weight.shape[0]
    assert n % 8 == 0 and c_in % _LANES == 0
    kj = c_in // _LANES
    tile_co = c_out // (2 * _N_SUB)
    assert tile_co % _LANES == 0

    x3 = x.reshape(n, kj, _LANES)
    w3 = weight.reshape(c_out, kj, _LANES)
    gamma2 = gamma.reshape(1, c_out).astype(jnp.float32)
    beta2 = beta.reshape(1, c_out).astype(jnp.float32)

    body = functools.partial(_bnneck_kernel, n=n)
    return pl.pallas_call(
        body,
        out_shape=jax.ShapeDtypeStruct((n, c_out), x.dtype),
        grid=(2, _N_SUB),
        in_specs=[
            pl.BlockSpec((n, kj, _LANES), lambda i, j: (0, 0, 0)),
            pl.BlockSpec((tile_co, kj, _LANES),
                         lambda i, j: (i * _N_SUB + j, 0, 0)),
            pl.BlockSpec((1, tile_co), lambda i, j: (0, i * _N_SUB + j)),
            pl.BlockSpec((1, tile_co), lambda i, j: (0, i * _N_SUB + j)),
        ],
        out_specs=pl.BlockSpec((n, tile_co), lambda i, j: (0, i * _N_SUB + j)),
        scratch_shapes=[pltpu.VMEM((n, c_in), jnp.bfloat16)],
        compiler_params=pltpu.CompilerParams(
            dimension_semantics=("parallel", "arbitrary"),
            vmem_limit_bytes=56 * 1024 * 1024,
        ),
    )(x3, w3, gamma2, beta2)


# speedup vs baseline: 1.0320x; 1.0320x over previous
import functools

import jax
import jax.numpy as jnp
from jax.experimental import pallas as pl
from jax.experimental.pallas import tpu as pltpu

_LANES = 128
_N_SUB = 2


def _bnneck_kernel(x_ref, w_ref, gamma_ref, beta_ref, o_ref, x_asm, *, n):
    c_in = x_ref.shape[1] * _LANES

    @pl.when(pl.program_id(1) == 0)
    def _cache_x():
        x_asm[...] = x_ref[...].astype(jnp.bfloat16).reshape(n, c_in)

    wk = w_ref[...].astype(jnp.bfloat16).reshape(w_ref.shape[0], c_in)
    y = jax.lax.dot_general(
        x_asm[...], wk, dimension_numbers=(((1,), (1,)), ((), ())),
        preferred_element_type=jnp.float32)
    inv_n = 1.0 / float(n)
    mean = jnp.sum(y, axis=0, keepdims=True) * inv_n
    diff = y - mean
    var = jnp.sum(diff * diff, axis=0, keepdims=True) * inv_n
    z = diff * jax.lax.rsqrt(var + 1e-5)
    z = z * gamma_ref[...] + beta_ref[...]
    o_ref[...] = jnp.where(z >= 0, z, 0.25 * z)


def kernel(x, weight, gamma, beta):
    n, c_in, h, w_sp = x.shape
    assert h == 1 and w_sp == 1
    c_out = weight.shape[0]
    assert n % 8 == 0 and c_in % _LANES == 0
    kj = c_in // _LANES
    tile_co = c_out // (2 * _N_SUB)
    assert tile_co % _LANES == 0

    x3 = x.reshape(n, kj, _LANES)
    w3 = weight.reshape(c_out, kj, _LANES)
    gamma2 = gamma.reshape(1, c_out).astype(jnp.float32)
    beta2 = beta.reshape(1, c_out).astype(jnp.float32)

    body = functools.partial(_bnneck_kernel, n=n)
    return pl.pallas_call(
        body,
        out_shape=jax.ShapeDtypeStruct((n, c_out), x.dtype),
        grid=(2, _N_SUB),
        in_specs=[
            pl.BlockSpec((n, kj, _LANES), lambda i, j: (0, 0, 0)),
            pl.BlockSpec((tile_co, kj, _LANES),
                         lambda i, j: (i * _N_SUB + j, 0, 0)),
            pl.BlockSpec((1, tile_co), lambda i, j: (0, i * _N_SUB + j)),
            pl.BlockSpec((1, tile_co), lambda i, j: (0, i * _N_SUB + j)),
        ],
        out_specs=pl.BlockSpec((n, tile_co), lambda i, j: (0, i * _N_SUB + j)),
        scratch_shapes=[pltpu.VMEM((n, c_in), jnp.bfloat16)],
        compiler_params=pltpu.CompilerParams(
            dimension_semantics=("parallel", "arbitrary"),
            vmem_limit_bytes=56 * 1024 * 1024,
        ),
    )(x3, w3, gamma2, beta2)
